# Initial kernel scaffold; baseline (speedup 1.0000x reference)
import jax
import jax.numpy as jnp
from jax.experimental import pallas as pl


def kernel(agts, ctx, agt_ctrs, ctx_ctrs, hi, wi, dist_w1, dist_b1, dist_w2, dist_gn_g, dist_gn_b, query_w, query_gn_g, query_gn_b, ctx_w1, ctx_gn_g, ctx_gn_b, ctx_w2, agt_w, norm_g, norm_b, lin_w, lin_gn_g, lin_gn_b):
    raise NotImplementedError("write your pallas kernel here")


# speedup vs baseline: 2.4841x; 2.4841x over previous
import functools

import jax
import jax.numpy as jnp
from jax import lax
from jax.experimental import pallas as pl
from jax.experimental.pallas import tpu as pltpu
from jax.experimental.pallas import tpu_sc as plsc

EPS = 1e-5

NC = 2
NS = 16
NW = NC * NS

CH = 80


def _gn(x, g, b):
    m = jnp.mean(x, axis=1, keepdims=True)
    d = x - m
    v = jnp.mean(d * d, axis=1, keepdims=True)
    return d * lax.rsqrt(v + EPS) * g + b


def _pre_body(agts, ctx, actrs, cctrs, w1t, b1, qwT, qg, qb, w1qT, w1xT, awT,
              ta_out, tc_out, a0_out):
    pa = (actrs[:, 0:1] * w1t[0:1, :] + actrs[:, 1:2] * w1t[1:2, :]) + b1[...]
    q = jax.nn.relu(_gn(jnp.dot(agts[...], qwT[...],
                                preferred_element_type=jnp.float32),
                        qg[...], qb[...]))
    qc = jnp.dot(q, w1qT[...], preferred_element_type=jnp.float32)
    ta_out[...] = jnp.concatenate([pa, qc], axis=1)

    pcn = -(cctrs[:, 0:1] * w1t[0:1, :] + cctrs[:, 1:2] * w1t[1:2, :])
    xc = jnp.dot(ctx[...], w1xT[...], preferred_element_type=jnp.float32)
    tc_out[...] = jnp.concatenate([pcn, xc], axis=1)

    a0_out[...] = jnp.dot(agts[...], awT[...],
                          preferred_element_type=jnp.float32)


def _run_pre(agts, ctx, actrs, cctrs, w1t, b1, qwT, qg, qb, w1qT, w1xT, awT,
             n_blk, interpret=False):
    n = agts.shape[0]
    d = agts.shape[1]
    grid = (n // n_blk,)
    full = lambda r, c: pl.BlockSpec((r, c), lambda i: (0, 0))
    row = lambda c: pl.BlockSpec((n_blk, c), lambda i: (i, 0))
    return pl.pallas_call(
        _pre_body,
        grid=grid,
        in_specs=[row(d), row(d), row(2), row(2), full(2, d), full(1, d),
                  full(d, d), full(1, d), full(1, d), full(d, d), full(d, d),
                  full(d, d)],
        out_specs=[row(2 * d), row(2 * d), row(d)],
        out_shape=[jax.ShapeDtypeStruct((n, 2 * d), jnp.float32),
                   jax.ShapeDtypeStruct((n, 2 * d), jnp.float32),
                   jax.ShapeDtypeStruct((n, d), jnp.float32)],
        interpret=interpret,
    )(agts, ctx, actrs, cctrs, w1t, b1, qwT, qg, qb, w1qT, w1xT, awT)


def _gather_body(nchunk, epw, ta_hbm, tc_hbm, hi_hbm, wi_hbm, g_hbm,
                 hi_v, wi_v, rows_a, rows_b, sem1, sem2):
    c = lax.axis_index("c")
    s = lax.axis_index("s")
    w = s * NC + c
    base = w * epw
    d2 = rows_a.shape[1]
    pltpu.sync_copy(hi_hbm.at[w], hi_v)
    pltpu.sync_copy(wi_hbm.at[w], wi_v)

    def body(j, carry):
        cpa = pltpu.async_copy(ta_hbm.at[hi_v.at[j]], rows_a, sem1)
        cpb = pltpu.async_copy(tc_hbm.at[wi_v.at[j]], rows_b, sem2)
        cpa.wait()
        cpb.wait()

        def add_row(r, cr):
            for k in range(d2 // 16):
                plsc.addupdate(rows_a.at[r, pl.ds(k * 16, 16)],
                               rows_b[r, pl.ds(k * 16, 16)])
            return cr

        lax.fori_loop(0, CH, add_row, 0)
        off = pl.multiple_of(base + j * CH, 8)
        pltpu.sync_copy(rows_a, g_hbm.at[pl.ds(off, CH)])
        return carry

    lax.fori_loop(0, nchunk, body, 0)


def _run_gather(ta, tc, hi3, wi3, e):
    d2 = ta.shape[1]
    epw = e // NW
    nchunk = epw // CH
    mesh = plsc.VectorSubcoreMesh(core_axis_name="c", subcore_axis_name="s")
    kern = pl.kernel(
        functools.partial(_gather_body, nchunk, epw),
        out_type=jax.ShapeDtypeStruct((e, d2), jnp.float32),
        mesh=mesh,
        scratch_types=[
            pltpu.VMEM((nchunk, CH), jnp.int32),
            pltpu.VMEM((nchunk, CH), jnp.int32),
            pltpu.VMEM((CH, d2), jnp.float32),
            pltpu.VMEM((CH, d2), jnp.float32),
            pltpu.SemaphoreType.DMA,
            pltpu.SemaphoreType.DMA,
        ],
    )
    return kern(ta, tc, hi3, wi3)


def _mlp_body(g_ref, w2T, dg, db, w1dT, cg, cb, cw2T, out_ref):
    d = out_ref.shape[1]
    gblk = g_ref[...]
    d1 = jax.nn.relu(gblk[:, :d])
    t = jnp.dot(d1, w2T[...], preferred_element_type=jnp.float32)
    d2 = jax.nn.relu(_gn(t, dg[...], db[...]))
    pre = jnp.dot(d2, w1dT[...], preferred_element_type=jnp.float32) \
        + gblk[:, d:]
    cc = jax.nn.relu(_gn(pre, cg[...], cb[...]))
    out_ref[...] = jnp.dot(cc, cw2T[...], preferred_element_type=jnp.float32)


def _run_mlp(g, w2T, dg, db, w1dT, cg, cb, cw2T, e_blk, interpret=False):
    e, d2 = g.shape
    d = d2 // 2
    grid = (e // e_blk,)
    full = lambda r, c: pl.BlockSpec((r, c), lambda i: (0, 0))
    return pl.pallas_call(
        _mlp_body,
        grid=grid,
        in_specs=[pl.BlockSpec((e_blk, d2), lambda i: (i, 0)),
                  full(d, d), full(1, d), full(1, d), full(d, d),
                  full(1, d), full(1, d), full(d, d)],
        out_specs=pl.BlockSpec((e_blk, d), lambda i: (i, 0)),
        out_shape=jax.ShapeDtypeStruct((e, d), jnp.float32),
        interpret=interpret,
    )(g, w2T, dg, db, w1dT, cg, cb, cw2T)


def _scatter_body(n, d, nchunk, epw, msg_hbm, hi_hbm, parts_hbm,
                  accum, hi_v, rows_v):
    c = lax.axis_index("c")
    s = lax.axis_index("s")
    w = s * NC + c
    base = w * epw
    n_node_chunks = n // CH
    my_chunks = (n_node_chunks - s + NS - 1) // NS

    def zb(i, carry):
        r = i // (d // 16)
        k = i % (d // 16)
        rows_v[r, pl.ds(k * 16, 16)] = jnp.zeros((16,), jnp.float32)
        return carry

    lax.fori_loop(0, CH * (d // 16), zb, 0)

    def zc(t, carry):
        zoff = pl.multiple_of((s + t * NS) * CH, 8)
        pltpu.sync_copy(rows_v, accum.at[pl.ds(zoff, CH)])
        return carry

    lax.fori_loop(0, my_chunks, zc, 0)
    plsc.subcore_barrier()

    pltpu.sync_copy(hi_hbm.at[w], hi_v)

    def body(j, carry):
        off = pl.multiple_of(base + j * CH, 8)
        pltpu.sync_copy(msg_hbm.at[pl.ds(off, CH)], rows_v)
        pltpu.sync_copy(rows_v, accum.at[hi_v.at[j]], add=True)
        return carry

    lax.fori_loop(0, nchunk, body, 0)
    plsc.subcore_barrier()

    def dump(t, carry):
        off = pl.multiple_of((s + t * NS) * CH, 8)
        pltpu.sync_copy(accum.at[pl.ds(off, CH)], rows_v)
        pltpu.sync_copy(rows_v, parts_hbm.at[c].at[pl.ds(off, CH)])
        return carry

    lax.fori_loop(0, my_chunks, dump, 0)


def _run_scatter(msg, hi3, n):
    e, d = msg.shape
    epw = e // NW
    nchunk = epw // CH
    mesh = plsc.VectorSubcoreMesh(core_axis_name="c", subcore_axis_name="s")
    kern = pl.kernel(
        functools.partial(_scatter_body, n, d, nchunk, epw),
        out_type=jax.ShapeDtypeStruct((NC, n, d), jnp.float32),
        mesh=mesh,
        scratch_types=[
            pltpu.VMEM_SHARED((n, d), jnp.float32),
            pltpu.VMEM((nchunk, CH), jnp.int32),
            pltpu.VMEM((CH, d), jnp.float32),
        ],
    )
    return kern(msg, hi3)


def _post_body(a0, parts, agts, ng, nb, lwT, lg, lb, out_ref):
    a = a0[...] + parts[0] + parts[1]
    a = jax.nn.relu(_gn(a, ng[...], nb[...]))
    b2 = jnp.dot(a, lwT[...], preferred_element_type=jnp.float32)
    b2 = _gn(b2, lg[...], lb[...])
    out_ref[...] = jax.nn.relu(b2 + agts[...])


def _run_post(a0, parts, agts, ng, nb, lwT, lg, lb, n_blk, interpret=False):
    n, d = a0.shape
    grid = (n // n_blk,)
    full = lambda r, c: pl.BlockSpec((r, c), lambda i: (0, 0))
    row = lambda c: pl.BlockSpec((n_blk, c), lambda i: (i, 0))
    return pl.pallas_call(
        _post_body,
        grid=grid,
        in_specs=[row(d), pl.BlockSpec((NC, n_blk, d), lambda i: (0, i, 0)),
                  row(d), full(1, d), full(1, d), full(d, d), full(1, d),
                  full(1, d)],
        out_specs=row(d),
        out_shape=jax.ShapeDtypeStruct((n, d), jnp.float32),
        interpret=interpret,
    )(a0, parts, agts, ng, nb, lwT, lg, lb)


def kernel(agts, ctx, agt_ctrs, ctx_ctrs, hi, wi, dist_w1, dist_b1, dist_w2,
           dist_gn_g, dist_gn_b, query_w, query_gn_g, query_gn_b, ctx_w1,
           ctx_gn_g, ctx_gn_b, ctx_w2, agt_w, norm_g, norm_b, lin_w,
           lin_gn_g, lin_gn_b):
    n, d = agts.shape
    e = hi.shape[0]
    epw = e // NW
    nchunk = epw // CH

    r1 = lambda x: x.reshape(1, d)
    ta, tc, a0 = _run_pre(
        agts, ctx, agt_ctrs, ctx_ctrs,
        dist_w1.T, dist_b1.reshape(1, d), query_w.T,
        r1(query_gn_g), r1(query_gn_b),
        ctx_w1[:, d:2 * d].T, ctx_w1[:, 2 * d:3 * d].T, agt_w.T,
        n_blk=1000)

    hi3 = hi.reshape(NW, nchunk, CH)
    wi3 = wi.reshape(NW, nchunk, CH)
    g = _run_gather(ta, tc, hi3, wi3, e)

    msg = _run_mlp(g, dist_w2.T, r1(dist_gn_g), r1(dist_gn_b),
                   ctx_w1[:, :d].T, r1(ctx_gn_g), r1(ctx_gn_b), ctx_w2.T,
                   e_blk=512)

    parts = _run_scatter(msg, hi3, n)

    return _run_post(a0, parts, agts, r1(norm_g), r1(norm_b), lin_w.T,
                     r1(lin_gn_g), r1(lin_gn_b), n_blk=1000)
